# baseline (speedup 1.0000x reference)
import functools

import jax
import jax.numpy as jnp
from jax import lax
from jax.experimental import pallas as pl
from jax.experimental.pallas import tpu as pltpu
from jax.experimental.pallas import tpu_sc as plsc

R, C = 128, 32768
NC, NS, L = 2, 16, 16
NW = NC * NS
R_SC = 32
R_TC = R - R_SC
K = 8
NCH = 4
CH = C // NCH
NVC = CH // L

_mesh = plsc.VectorSubcoreMesh(core_axis_name="c", subcore_axis_name="s")


def _merge(a, b):
    va, ca = a
    vb, cb = b
    take = (vb > va) | ((vb == va) & (cb < ca))
    return jnp.where(take, vb, va), jnp.where(take, cb, ca)


@functools.partial(
    pl.kernel,
    out_type=jax.ShapeDtypeStruct((NW, L), jnp.int32),
    mesh=_mesh,
    scratch_types=[
        pltpu.VMEM((CH,), jnp.float32),
        pltpu.VMEM((CH,), jnp.float32),
        pltpu.VMEM((L,), jnp.int32),
        pltpu.SemaphoreType.DMA,
        pltpu.SemaphoreType.DMA,
    ],
)
def _argmax_sc(x_hbm, out_hbm, buf_a, buf_b, res_v, sem_a, sem_b):
    wid = lax.axis_index("s") * NC + lax.axis_index("c")
    row = R_TC + wid
    bufs = (buf_a, buf_b)
    sems = (sem_a, sem_b)
    lane = jnp.arange(L, dtype=jnp.int32)

    copies = [None, None]
    copies[0] = pltpu.async_copy(x_hbm.at[row, pl.ds(0, CH)], buf_a, sem_a)

    best = (jnp.full((L,), -jnp.inf, jnp.float32), jnp.zeros((L,), jnp.int32))
    for ch in range(NCH):
        buf = bufs[ch % 2]
        copies[ch % 2].wait()
        if ch + 1 < NCH:
            copies[(ch + 1) % 2] = pltpu.async_copy(
                x_hbm.at[row, pl.ds((ch + 1) * CH, CH)],
                bufs[(ch + 1) % 2],
                sems[(ch + 1) % 2],
            )

        init = tuple(
            (jnp.full((L,), -jnp.inf, jnp.float32), jnp.zeros((L,), jnp.int32))
            for _ in range(K)
        )

        def body(i, accs, buf=buf):
            base = i * (K * L)
            ib = jnp.full((L,), i, jnp.int32)
            out = []
            for u in range(K):
                vmax, vidx = accs[u]
                xv = buf[pl.ds(base + u * L, L)]
                gt = xv > vmax
                out.append((jnp.where(gt, xv, vmax), jnp.where(gt, ib, vidx)))
            return tuple(out)

        accs = plsc.parallel_loop(0, NVC // K, carry=init, unroll=2)(body)

        pairs = [
            (vmax, vidx * (K * L) + u * L + lane + ch * CH)
            for u, (vmax, vidx) in enumerate(accs)
        ]
        while len(pairs) > 1:
            pairs = [_merge(pairs[j], pairs[j + 1]) for j in range(0, len(pairs), 2)]
        best = _merge(best, pairs[0])

    vals, idxs = best
    for sh in (8, 4, 2, 1):
        perm = lane ^ sh
        ov = vals.at[perm].get(mode="promise_in_bounds")
        oi = idxs.at[perm].get(mode="promise_in_bounds")
        vals, idxs = _merge((vals, idxs), (ov, oi))

    res_v[...] = idxs
    pltpu.sync_copy(res_v, out_hbm.at[wid])


RB = 8
CB = 2048
NCB = C // CB
NRB = R_TC // RB


def _tc_body(x_ref, o_ref, vmax_ref, vidx_ref):
    ci = pl.program_id(1)

    @pl.when(ci == 0)
    def _():
        vmax_ref[...] = jnp.full((RB, 128), -jnp.inf, jnp.float32)
        vidx_ref[...] = jnp.zeros((RB, 128), jnp.int32)

    vmax = vmax_ref[...]
    vidx = vidx_ref[...]
    lane = lax.broadcasted_iota(jnp.int32, (RB, 128), 1)
    base = ci * CB
    for k in range(CB // 128):
        xv = x_ref[:, k * 128:(k + 1) * 128]
        gt = xv > vmax
        vmax = jnp.where(gt, xv, vmax)
        vidx = jnp.where(gt, base + k * 128 + lane, vidx)
    vmax_ref[...] = vmax
    vidx_ref[...] = vidx

    @pl.when(ci == NCB - 1)
    def _():
        rowmax = jnp.max(vmax, axis=1, keepdims=True)
        cand = jnp.where(vmax == rowmax, vidx, jnp.int32(2**31 - 1))
        argcol = jnp.min(cand, axis=1, keepdims=True)
        o_ref[...] = jnp.broadcast_to(argcol, (RB, 128))[None]


_argmax_tc = pl.pallas_call(
    _tc_body,
    grid=(NRB, NCB),
    in_specs=[pl.BlockSpec((RB, CB), lambda i, ci: (i, ci))],
    out_specs=pl.BlockSpec((1, RB, 128), lambda i, ci: (i, 0, 0)),
    out_shape=jax.ShapeDtypeStruct((NRB, RB, 128), jnp.int32),
    scratch_shapes=[
        pltpu.VMEM((RB, 128), jnp.float32),
        pltpu.VMEM((RB, 128), jnp.int32),
    ],
    compiler_params=pltpu.CompilerParams(
        dimension_semantics=("parallel", "arbitrary"),
    ),
)


def kernel(x):
    sc_out = _argmax_sc(x)
    tc_out = _argmax_tc(x[:R_TC])
    return jnp.concatenate([tc_out[:, :, 0].reshape(R_TC), sc_out[:, 0]])


# speedup vs baseline: 1.7469x; 1.7469x over previous
import functools

import jax
import jax.numpy as jnp
from jax import lax
from jax.experimental import pallas as pl
from jax.experimental.pallas import tpu as pltpu
from jax.experimental.pallas import tpu_sc as plsc

R, C = 128, 32768
NC, NS, L = 2, 16, 16
NW = NC * NS
R_SC = 32
R_TC = R - R_SC
K = 8
NCH = 4
CH = C // NCH
NVC = CH // L

_mesh = plsc.VectorSubcoreMesh(core_axis_name="c", subcore_axis_name="s")


def _merge(a, b):
    va, ca = a
    vb, cb = b
    take = (vb > va) | ((vb == va) & (cb < ca))
    return jnp.where(take, vb, va), jnp.where(take, cb, ca)


@functools.partial(
    pl.kernel,
    out_type=jax.ShapeDtypeStruct((NW, L), jnp.int32),
    mesh=_mesh,
    scratch_types=[
        pltpu.VMEM((CH,), jnp.float32),
        pltpu.VMEM((CH,), jnp.float32),
        pltpu.VMEM((L,), jnp.int32),
        pltpu.SemaphoreType.DMA,
        pltpu.SemaphoreType.DMA,
    ],
)
def _argmax_sc(x_hbm, out_hbm, buf_a, buf_b, res_v, sem_a, sem_b):
    wid = lax.axis_index("s") * NC + lax.axis_index("c")
    row = R_TC + wid
    bufs = (buf_a, buf_b)
    sems = (sem_a, sem_b)
    lane = jnp.arange(L, dtype=jnp.int32)

    copies = [None, None]
    copies[0] = pltpu.async_copy(x_hbm.at[row, pl.ds(0, CH)], buf_a, sem_a)

    best = (jnp.full((L,), -jnp.inf, jnp.float32), jnp.zeros((L,), jnp.int32))
    for ch in range(NCH):
        buf = bufs[ch % 2]
        copies[ch % 2].wait()
        if ch + 1 < NCH:
            copies[(ch + 1) % 2] = pltpu.async_copy(
                x_hbm.at[row, pl.ds((ch + 1) * CH, CH)],
                bufs[(ch + 1) % 2],
                sems[(ch + 1) % 2],
            )

        init = tuple(
            (jnp.full((L,), -jnp.inf, jnp.float32), jnp.zeros((L,), jnp.int32))
            for _ in range(K)
        )

        def body(i, accs, buf=buf):
            base = i * (K * L)
            ib = jnp.full((L,), i, jnp.int32)
            out = []
            for u in range(K):
                vmax, vidx = accs[u]
                xv = buf[pl.ds(base + u * L, L)]
                gt = xv > vmax
                out.append((jnp.where(gt, xv, vmax), jnp.where(gt, ib, vidx)))
            return tuple(out)

        accs = plsc.parallel_loop(0, NVC // K, carry=init, unroll=2)(body)

        pairs = [
            (vmax, vidx * (K * L) + u * L + lane + ch * CH)
            for u, (vmax, vidx) in enumerate(accs)
        ]
        while len(pairs) > 1:
            pairs = [_merge(pairs[j], pairs[j + 1]) for j in range(0, len(pairs), 2)]
        best = _merge(best, pairs[0])

    vals, idxs = best
    for sh in (8, 4, 2, 1):
        perm = lane ^ sh
        ov = vals.at[perm].get(mode="promise_in_bounds")
        oi = idxs.at[perm].get(mode="promise_in_bounds")
        vals, idxs = _merge((vals, idxs), (ov, oi))

    res_v[...] = idxs
    pltpu.sync_copy(res_v, out_hbm.at[wid])


RB = 8
CB = 4096
NCB = C // CB
NRB = R_TC // RB


def _tc_body(x_ref, o_ref, vmax_ref, vblk_ref):
    ci = pl.program_id(1)
    xv = x_ref[...]

    @pl.when(ci == 0)
    def _():
        vmax_ref[...] = xv
        vblk_ref[...] = jnp.zeros((RB, CB), jnp.int32)

    @pl.when(ci > 0)
    def _():
        vmax = vmax_ref[...]
        gt = xv > vmax
        vmax_ref[...] = jnp.where(gt, xv, vmax)
        vblk_ref[...] = jnp.where(gt, ci, vblk_ref[...])

    @pl.when(ci == NCB - 1)
    def _():
        vmax = vmax_ref[...]
        cols = vblk_ref[...] * CB + lax.broadcasted_iota(jnp.int32, (RB, CB), 1)
        rowmax = jnp.max(vmax, axis=1, keepdims=True)
        cand = jnp.where(vmax == rowmax, cols, jnp.int32(2**31 - 1))
        argcol = jnp.min(cand, axis=1, keepdims=True)
        o_ref[...] = jnp.broadcast_to(argcol, (RB, 128))[None]


_argmax_tc = pl.pallas_call(
    _tc_body,
    grid=(NRB, NCB),
    in_specs=[pl.BlockSpec((RB, CB), lambda i, ci: (i, ci))],
    out_specs=pl.BlockSpec((1, RB, 128), lambda i, ci: (i, 0, 0)),
    out_shape=jax.ShapeDtypeStruct((NRB, RB, 128), jnp.int32),
    scratch_shapes=[
        pltpu.VMEM((RB, CB), jnp.float32),
        pltpu.VMEM((RB, CB), jnp.int32),
    ],
    compiler_params=pltpu.CompilerParams(
        dimension_semantics=("parallel", "arbitrary"),
    ),
)


def kernel(x):
    sc_out = _argmax_sc(x)
    tc_out = _argmax_tc(x)
    return jnp.concatenate([tc_out[:, :, 0].reshape(R_TC), sc_out[:, 0]])


# speedup vs baseline: 4.0051x; 2.2927x over previous
import functools

import jax
import jax.numpy as jnp
from jax import lax
from jax.experimental import pallas as pl
from jax.experimental.pallas import tpu as pltpu
from jax.experimental.pallas import tpu_sc as plsc

R, C = 128, 32768
NC, NS, L = 2, 16, 16
NW = NC * NS
R_SC = 32
R_TC = R - R_SC
K = 8
NCH = 4
CH = C // NCH
NVC = CH // L

_mesh = plsc.VectorSubcoreMesh(core_axis_name="c", subcore_axis_name="s")


def _merge(a, b):
    va, ca = a
    vb, cb = b
    take = (vb > va) | ((vb == va) & (cb < ca))
    return jnp.where(take, vb, va), jnp.where(take, cb, ca)


@functools.partial(
    pl.kernel,
    out_type=jax.ShapeDtypeStruct((NW, L), jnp.int32),
    mesh=_mesh,
    scratch_types=[
        pltpu.VMEM((CH,), jnp.float32),
        pltpu.VMEM((CH,), jnp.float32),
        pltpu.VMEM((L,), jnp.int32),
        pltpu.SemaphoreType.DMA,
        pltpu.SemaphoreType.DMA,
    ],
)
def _argmax_sc(x_hbm, out_hbm, buf_a, buf_b, res_v, sem_a, sem_b):
    wid = lax.axis_index("s") * NC + lax.axis_index("c")
    row = R_TC + wid
    bufs = (buf_a, buf_b)
    sems = (sem_a, sem_b)
    lane = jnp.arange(L, dtype=jnp.int32)

    copies = [None, None]
    copies[0] = pltpu.async_copy(x_hbm.at[row, pl.ds(0, CH)], buf_a, sem_a)

    best = (jnp.full((L,), -jnp.inf, jnp.float32), jnp.zeros((L,), jnp.int32))
    for ch in range(NCH):
        buf = bufs[ch % 2]
        copies[ch % 2].wait()
        if ch + 1 < NCH:
            copies[(ch + 1) % 2] = pltpu.async_copy(
                x_hbm.at[row, pl.ds((ch + 1) * CH, CH)],
                bufs[(ch + 1) % 2],
                sems[(ch + 1) % 2],
            )

        init = tuple(
            (jnp.full((L,), -jnp.inf, jnp.float32), jnp.zeros((L,), jnp.int32))
            for _ in range(K)
        )

        def body(i, accs, buf=buf):
            base = i * (K * L)
            ib = jnp.full((L,), i, jnp.int32)
            out = []
            for u in range(K):
                vmax, vidx = accs[u]
                xv = buf[pl.ds(base + u * L, L)]
                gt = xv > vmax
                out.append((jnp.where(gt, xv, vmax), jnp.where(gt, ib, vidx)))
            return tuple(out)

        accs = plsc.parallel_loop(0, NVC // K, carry=init, unroll=2)(body)

        pairs = [
            (vmax, vidx * (K * L) + u * L + lane + ch * CH)
            for u, (vmax, vidx) in enumerate(accs)
        ]
        while len(pairs) > 1:
            pairs = [_merge(pairs[j], pairs[j + 1]) for j in range(0, len(pairs), 2)]
        best = _merge(best, pairs[0])

    vals, idxs = best
    for sh in (8, 4, 2, 1):
        perm = lane ^ sh
        ov = vals.at[perm].get(mode="promise_in_bounds")
        oi = idxs.at[perm].get(mode="promise_in_bounds")
        vals, idxs = _merge((vals, idxs), (ov, oi))

    res_v[...] = idxs
    pltpu.sync_copy(res_v, out_hbm.at[wid])


RB = 8
NG = 16
GW = NG * 128
NOC = C // GW
NRB = R_TC // RB


def _tc_body(x_ref, o_ref):
    lane = lax.broadcasted_iota(jnp.int32, (RB, 128), 1)
    accs = [
        (jnp.full((RB, 128), -jnp.inf, jnp.float32), jnp.zeros((RB, 128), jnp.int32))
        for _ in range(NG)
    ]
    for c in range(NOC):
        for k in range(NG):
            xv = x_ref[:, c * GW + k * 128:c * GW + (k + 1) * 128]
            vmax, vc = accs[k]
            gt = xv > vmax
            accs[k] = (jnp.where(gt, xv, vmax), jnp.where(gt, c, vc))

    pairs = [(v, vc * GW + k * 128 + lane) for k, (v, vc) in enumerate(accs)]
    while len(pairs) > 1:
        pairs = [_merge(pairs[j], pairs[j + 1]) for j in range(0, len(pairs), 2)]
    vals, cols = pairs[0]
    rowmax = jnp.max(vals, axis=1, keepdims=True)
    cand = jnp.where(vals == rowmax, cols, jnp.int32(2**31 - 1))
    argcol = jnp.min(cand, axis=1, keepdims=True)
    o_ref[...] = jnp.broadcast_to(argcol, (RB, 128))[None]


_argmax_tc = pl.pallas_call(
    _tc_body,
    grid=(NRB,),
    in_specs=[pl.BlockSpec((RB, C), lambda i: (i, 0))],
    out_specs=pl.BlockSpec((1, RB, 128), lambda i: (i, 0, 0)),
    out_shape=jax.ShapeDtypeStruct((NRB, RB, 128), jnp.int32),
    compiler_params=pltpu.CompilerParams(
        dimension_semantics=("parallel",),
    ),
)


def kernel(x):
    sc_out = _argmax_sc(x)
    tc_out = _argmax_tc(x)
    return jnp.concatenate([tc_out[:, :, 0].reshape(R_TC), sc_out[:, 0]])


# speedup vs baseline: 4.2433x; 1.0595x over previous
import functools

import jax
import jax.numpy as jnp
from jax import lax
from jax.experimental import pallas as pl
from jax.experimental.pallas import tpu as pltpu
from jax.experimental.pallas import tpu_sc as plsc

R, C = 128, 32768
NC, NS, L = 2, 16, 16
NW = NC * NS
R_SC = 64
R_TC = R - R_SC
RPW = R_SC // NW
K = 8
NCH = 2
CH = C // NCH
NVC = CH // L

_mesh = plsc.VectorSubcoreMesh(core_axis_name="c", subcore_axis_name="s")


def _merge(a, b):
    va, ca = a
    vb, cb = b
    take = (vb > va) | ((vb == va) & (cb < ca))
    return jnp.where(take, vb, va), jnp.where(take, cb, ca)


@functools.partial(
    pl.kernel,
    out_type=jax.ShapeDtypeStruct((NW, L), jnp.int32),
    mesh=_mesh,
    scratch_types=[
        pltpu.VMEM((CH,), jnp.float32),
        pltpu.VMEM((CH,), jnp.float32),
        pltpu.VMEM((L,), jnp.int32),
        pltpu.SemaphoreType.DMA,
        pltpu.SemaphoreType.DMA,
    ],
)
def _argmax_sc(x_hbm, out_hbm, buf_a, buf_b, res_v, sem_a, sem_b):
    wid = lax.axis_index("s") * NC + lax.axis_index("c")
    row0 = R_TC + wid * RPW
    bufs = (buf_a, buf_b)
    sems = (sem_a, sem_b)
    lane = jnp.arange(L, dtype=jnp.int32)

    NT = RPW * NCH
    copies = [None, None]
    copies[0] = pltpu.async_copy(x_hbm.at[row0, pl.ds(0, CH)], buf_a, sem_a)

    resvec = jnp.zeros((L,), jnp.int32)
    best = (jnp.full((L,), -jnp.inf, jnp.float32), jnp.zeros((L,), jnp.int32))
    for t in range(NT):
        r, ch = divmod(t, NCH)
        buf = bufs[t % 2]
        copies[t % 2].wait()
        if t + 1 < NT:
            rn, chn = divmod(t + 1, NCH)
            copies[(t + 1) % 2] = pltpu.async_copy(
                x_hbm.at[row0 + rn, pl.ds(chn * CH, CH)],
                bufs[(t + 1) % 2],
                sems[(t + 1) % 2],
            )

        init = tuple(
            (jnp.full((L,), -jnp.inf, jnp.float32), jnp.zeros((L,), jnp.int32))
            for _ in range(K)
        )

        def body(i, accs, buf=buf):
            base = i * (K * L)
            ib = jnp.full((L,), i, jnp.int32)
            out = []
            for u in range(K):
                vmax, vidx = accs[u]
                xv = buf[pl.ds(base + u * L, L)]
                gt = xv > vmax
                out.append((jnp.where(gt, xv, vmax), jnp.where(gt, ib, vidx)))
            return tuple(out)

        accs = plsc.parallel_loop(0, NVC // K, carry=init, unroll=2)(body)

        pairs = [
            (vmax, vidx * (K * L) + u * L + lane + ch * CH)
            for u, (vmax, vidx) in enumerate(accs)
        ]
        while len(pairs) > 1:
            pairs = [_merge(pairs[j], pairs[j + 1]) for j in range(0, len(pairs), 2)]
        best = _merge(best, pairs[0])

        if ch == NCH - 1:
            vals, idxs = best
            for sh in (8, 4, 2, 1):
                perm = lane ^ sh
                ov = vals.at[perm].get(mode="promise_in_bounds")
                oi = idxs.at[perm].get(mode="promise_in_bounds")
                vals, idxs = _merge((vals, idxs), (ov, oi))
            resvec = jnp.where(lane == r, idxs, resvec)
            best = (
                jnp.full((L,), -jnp.inf, jnp.float32),
                jnp.zeros((L,), jnp.int32),
            )

    res_v[...] = resvec
    pltpu.sync_copy(res_v, out_hbm.at[wid])


RB = 8
NG = 16
GW = NG * 128
NOC = C // GW
NRB = R_TC // RB


def _tc_body(x_ref, o_ref):
    lane = lax.broadcasted_iota(jnp.int32, (RB, 128), 1)
    accs = [
        (jnp.full((RB, 128), -jnp.inf, jnp.float32), jnp.zeros((RB, 128), jnp.int32))
        for _ in range(NG)
    ]
    for c in range(NOC):
        for k in range(NG):
            xv = x_ref[:, c * GW + k * 128:c * GW + (k + 1) * 128]
            vmax, vc = accs[k]
            gt = xv > vmax
            accs[k] = (jnp.where(gt, xv, vmax), jnp.where(gt, c, vc))

    pairs = [(v, vc * GW + k * 128 + lane) for k, (v, vc) in enumerate(accs)]
    while len(pairs) > 1:
        pairs = [_merge(pairs[j], pairs[j + 1]) for j in range(0, len(pairs), 2)]
    vals, cols = pairs[0]
    rowmax = jnp.max(vals, axis=1, keepdims=True)
    cand = jnp.where(vals == rowmax, cols, jnp.int32(2**31 - 1))
    argcol = jnp.min(cand, axis=1, keepdims=True)
    o_ref[...] = jnp.broadcast_to(argcol, (RB, 128))[None]


_argmax_tc = pl.pallas_call(
    _tc_body,
    grid=(NRB,),
    in_specs=[pl.BlockSpec((RB, C), lambda i: (i, 0))],
    out_specs=pl.BlockSpec((1, RB, 128), lambda i: (i, 0, 0)),
    out_shape=jax.ShapeDtypeStruct((NRB, RB, 128), jnp.int32),
    compiler_params=pltpu.CompilerParams(
        dimension_semantics=("parallel",),
    ),
)


def kernel(x):
    sc_out = _argmax_sc(x)
    tc_out = _argmax_tc(x)
    return jnp.concatenate(
        [tc_out[:, :, 0].reshape(R_TC), sc_out[:, :RPW].reshape(R_SC)]
    )


# speedup vs baseline: 4.3489x; 1.0249x over previous
import functools

import jax
import jax.numpy as jnp
from jax import lax
from jax.experimental import pallas as pl
from jax.experimental.pallas import tpu as pltpu
from jax.experimental.pallas import tpu_sc as plsc

R, C = 128, 32768
NC, NS, L = 2, 16, 16
NW = NC * NS
R_SC = 64
R_TC = R - R_SC
RPW = R_SC // NW
K = 16
NCH = 1
CH = C // NCH
NVC = CH // L

_mesh = plsc.VectorSubcoreMesh(core_axis_name="c", subcore_axis_name="s")


def _merge(a, b):
    va, ca = a
    vb, cb = b
    take = (vb > va) | ((vb == va) & (cb < ca))
    return jnp.where(take, vb, va), jnp.where(take, cb, ca)


@functools.partial(
    pl.kernel,
    out_type=jax.ShapeDtypeStruct((NW, L), jnp.int32),
    mesh=_mesh,
    scratch_types=[
        pltpu.VMEM((CH,), jnp.float32),
        pltpu.VMEM((CH,), jnp.float32),
        pltpu.VMEM((L,), jnp.int32),
        pltpu.SemaphoreType.DMA,
        pltpu.SemaphoreType.DMA,
    ],
)
def _argmax_sc(x_hbm, out_hbm, buf_a, buf_b, res_v, sem_a, sem_b):
    wid = lax.axis_index("s") * NC + lax.axis_index("c")
    row0 = R_TC + wid * RPW
    bufs = (buf_a, buf_b)
    sems = (sem_a, sem_b)
    lane = jnp.arange(L, dtype=jnp.int32)

    NT = RPW * NCH
    copies = [None, None]
    copies[0] = pltpu.async_copy(x_hbm.at[row0, pl.ds(0, CH)], buf_a, sem_a)

    resvec = jnp.zeros((L,), jnp.int32)
    best = (jnp.full((L,), -jnp.inf, jnp.float32), jnp.zeros((L,), jnp.int32))
    for t in range(NT):
        r, ch = divmod(t, NCH)
        buf = bufs[t % 2]
        copies[t % 2].wait()
        if t + 1 < NT:
            rn, chn = divmod(t + 1, NCH)
            copies[(t + 1) % 2] = pltpu.async_copy(
                x_hbm.at[row0 + rn, pl.ds(chn * CH, CH)],
                bufs[(t + 1) % 2],
                sems[(t + 1) % 2],
            )

        init = tuple(
            (jnp.full((L,), -jnp.inf, jnp.float32), jnp.zeros((L,), jnp.int32))
            for _ in range(K)
        )

        def body(i, accs, buf=buf):
            base = i * (K * L)
            ib = jnp.full((L,), i, jnp.int32)
            out = []
            for u in range(K):
                vmax, vidx = accs[u]
                xv = buf[pl.ds(base + u * L, L)]
                gt = xv > vmax
                out.append((jnp.where(gt, xv, vmax), jnp.where(gt, ib, vidx)))
            return tuple(out)

        accs = plsc.parallel_loop(0, NVC // K, carry=init, unroll=2)(body)

        pairs = [
            (vmax, vidx * (K * L) + u * L + lane + ch * CH)
            for u, (vmax, vidx) in enumerate(accs)
        ]
        while len(pairs) > 1:
            pairs = [_merge(pairs[j], pairs[j + 1]) for j in range(0, len(pairs), 2)]
        best = _merge(best, pairs[0])

        if ch == NCH - 1:
            vals, idxs = best
            for sh in (8, 4, 2, 1):
                perm = lane ^ sh
                ov = vals.at[perm].get(mode="promise_in_bounds")
                oi = idxs.at[perm].get(mode="promise_in_bounds")
                vals, idxs = _merge((vals, idxs), (ov, oi))
            resvec = jnp.where(lane == r, idxs, resvec)
            best = (
                jnp.full((L,), -jnp.inf, jnp.float32),
                jnp.zeros((L,), jnp.int32),
            )

    res_v[...] = resvec
    pltpu.sync_copy(res_v, out_hbm.at[wid])


RB = 8
NG = 16
GW = NG * 128
NOC = C // GW
NRB = R_TC // RB


def _tc_body(x_ref, o_ref):
    lane = lax.broadcasted_iota(jnp.int32, (RB, 128), 1)
    accs = [
        (jnp.full((RB, 128), -jnp.inf, jnp.float32), jnp.zeros((RB, 128), jnp.int32))
        for _ in range(NG)
    ]
    for c in range(NOC):
        for k in range(NG):
            xv = x_ref[:, c * GW + k * 128:c * GW + (k + 1) * 128]
            vmax, vc = accs[k]
            gt = xv > vmax
            accs[k] = (jnp.where(gt, xv, vmax), jnp.where(gt, c, vc))

    pairs = [(v, vc * GW + k * 128 + lane) for k, (v, vc) in enumerate(accs)]
    while len(pairs) > 1:
        pairs = [_merge(pairs[j], pairs[j + 1]) for j in range(0, len(pairs), 2)]
    vals, cols = pairs[0]
    rowmax = jnp.max(vals, axis=1, keepdims=True)
    cand = jnp.where(vals == rowmax, cols, jnp.int32(2**31 - 1))
    argcol = jnp.min(cand, axis=1, keepdims=True)
    o_ref[...] = jnp.broadcast_to(argcol, (RB, 128))[None]


_argmax_tc = pl.pallas_call(
    _tc_body,
    grid=(NRB,),
    in_specs=[pl.BlockSpec((RB, C), lambda i: (i, 0))],
    out_specs=pl.BlockSpec((1, RB, 128), lambda i: (i, 0, 0)),
    out_shape=jax.ShapeDtypeStruct((NRB, RB, 128), jnp.int32),
    compiler_params=pltpu.CompilerParams(
        dimension_semantics=("parallel",),
    ),
)


def kernel(x):
    sc_out = _argmax_sc(x)
    tc_out = _argmax_tc(x)
    return jnp.concatenate(
        [tc_out[:, :, 0].reshape(R_TC), sc_out[:, :RPW].reshape(R_SC)]
    )
